# Initial kernel scaffold; baseline (speedup 1.0000x reference)
import jax
import jax.numpy as jnp
from jax.experimental import pallas as pl


def kernel(x, edge_index, edge_attr, batch, Wn1, bn1, Wn2, bn2, We1, be1, We2, be2, Wf, bf):
    raise NotImplementedError("write your pallas kernel here")


# speedup vs baseline: 79.5677x; 79.5677x over previous
import functools

import jax
import jax.numpy as jnp
from jax import lax
from jax.experimental import pallas as pl
from jax.experimental.pallas import tpu as pltpu
from jax.experimental.pallas import tpu_sc as plsc

_N = 10000
_E = 320000
_G = 64
_C = 80
_NW = 32
_NCH = _E // _C
_KW = _NCH // _NW
_BLK = 1000
_NB = _N // _BLK
_HB = (_E - _N) // _BLK

_f32 = jnp.float32


def _sc_mesh():
    return plsc.VectorSubcoreMesh(core_axis_name="c", subcore_axis_name="s")


def _sc_pre():
    def body(src_hbm, dst_hbm, batch_hbm, zer_hbm, deg_hbm, eseg_hbm,
             src_v, dst_v, seg_v, fbuf_v, ones_v, bat_sh, dacc_sh, sem):
        cid = lax.axis_index("c")
        sid = lax.axis_index("s")
        wid = sid * 2 + cid
        pltpu.sync_copy(src_hbm.at[wid], src_v)
        pltpu.sync_copy(dst_hbm.at[wid], dst_v)
        for i in range(_C // 16):
            ones_v[pl.ds(i * 16, 16)] = jnp.ones((16,), _f32)

        for j in range(8):
            c = pl.multiple_of((sid * 8 + j) * _C, _C)

            @pl.when(sid * 8 + j < 125)
            def _():
                pltpu.sync_copy(batch_hbm.at[pl.ds(c, _C)], seg_v)
                pltpu.sync_copy(seg_v, bat_sh.at[pl.ds(c, _C)])
                pltpu.sync_copy(zer_hbm.at[pl.ds(c, _C)], fbuf_v)
                pltpu.sync_copy(fbuf_v, dacc_sh.at[pl.ds(c, _C)])

        plsc.subcore_barrier()

        def step(k, carry):
            pltpu.async_copy(bat_sh.at[src_v.at[k]], seg_v, sem).wait()
            off = pl.multiple_of((wid * _KW + k) * _C, _C)
            pltpu.sync_copy(seg_v, eseg_hbm.at[pl.ds(off, _C)])
            pltpu.sync_copy(ones_v, dacc_sh.at[dst_v.at[k]], add=True)
            return carry

        lax.fori_loop(0, _KW, step, 0)
        plsc.subcore_barrier()

        for j in range(8):
            c = pl.multiple_of((sid * 8 + j) * _C, _C)
            o = pl.multiple_of(cid * _N + (sid * 8 + j) * _C, _C)

            @pl.when(sid * 8 + j < 125)
            def _():
                pltpu.sync_copy(dacc_sh.at[pl.ds(c, _C)], fbuf_v)
                pltpu.sync_copy(fbuf_v, deg_hbm.at[pl.ds(o, _C)])

    return pl.kernel(
        body,
        out_type=[jax.ShapeDtypeStruct((2 * _N,), _f32),
                  jax.ShapeDtypeStruct((_E,), jnp.int32)],
        mesh=_sc_mesh(),
        compiler_params=pltpu.CompilerParams(use_tc_tiling_on_sc=False),
        scratch_types=[
            pltpu.VMEM((_KW, _C), jnp.int32),
            pltpu.VMEM((_KW, _C), jnp.int32),
            pltpu.VMEM((_C,), jnp.int32),
            pltpu.VMEM((_C,), _f32),
            pltpu.VMEM((_C,), _f32),
            pltpu.VMEM_SHARED((_N,), jnp.int32),
            pltpu.VMEM_SHARED((_N,), _f32),
            pltpu.SemaphoreType.DMA,
        ],
    )


def _sc_agg(d):
    def body(src_hbm, dst_hbm, tab_hbm, zer_hbm, out_hbm,
             src_v, dst_v, rows_v, acc_sh, sem):
        cid = lax.axis_index("c")
        sid = lax.axis_index("s")
        wid = sid * 2 + cid
        pltpu.sync_copy(src_hbm.at[wid], src_v)
        pltpu.sync_copy(dst_hbm.at[wid], dst_v)

        for j in range(8):
            c = pl.multiple_of((sid * 8 + j) * _C, _C)

            @pl.when(sid * 8 + j < 125)
            def _():
                pltpu.sync_copy(zer_hbm.at[pl.ds(c, _C)], rows_v)
                pltpu.sync_copy(rows_v, acc_sh.at[pl.ds(c, _C)])

        plsc.subcore_barrier()

        def step(k, carry):
            pltpu.async_copy(tab_hbm.at[src_v.at[k]], rows_v, sem).wait()
            pltpu.sync_copy(rows_v, acc_sh.at[dst_v.at[k]], add=True)
            return carry

        lax.fori_loop(0, _KW, step, 0)
        plsc.subcore_barrier()

        for j in range(8):
            c = pl.multiple_of((sid * 8 + j) * _C, _C)

            @pl.when(sid * 8 + j < 125)
            def _():
                pltpu.sync_copy(acc_sh.at[pl.ds(c, _C)], rows_v)
                pltpu.sync_copy(rows_v, out_hbm.at[cid, pl.ds(c, _C)])

    return pl.kernel(
        body,
        out_type=jax.ShapeDtypeStruct((2, _N, d), _f32),
        mesh=_sc_mesh(),
        compiler_params=pltpu.CompilerParams(use_tc_tiling_on_sc=False),
        scratch_types=[
            pltpu.VMEM((_KW, _C), jnp.int32),
            pltpu.VMEM((_KW, _C), jnp.int32),
            pltpu.VMEM((_C, d), _f32),
            pltpu.VMEM_SHARED((_N, d), _f32),
            pltpu.SemaphoreType.DMA,
        ],
    )


def _dinv_of(degt):
    return lax.rsqrt(1.0 + degt[:, 0:1] + degt[:, 1:2])


def _a_body(degt_ref, x_ref, ea_ref, xp_ref, eap_ref):
    dinv = _dinv_of(degt_ref[...])
    xp_ref[...] = x_ref[...] * dinv
    eap_ref[...] = ea_ref[...] * dinv


def _c_body(s_ref, xp_ref, degt_ref, w1_ref, b1_ref, w2_ref, out_ref):
    dinv = _dinv_of(degt_ref[...])
    z = (s_ref[0] + s_ref[1] + xp_ref[...]) * dinv
    h = jax.nn.sigmoid(jnp.dot(z, w1_ref[...], preferred_element_type=_f32)
                       + b1_ref[...])
    out_ref[...] = jnp.dot(h, w2_ref[...], preferred_element_type=_f32) * dinv


def _pool(seg, feat, sum_ref, cnt_ref):
    oh = (lax.broadcasted_iota(jnp.int32, (_G, seg.shape[0]), 0)
          == seg[None, :]).astype(_f32)
    sum_ref[...] += jnp.dot(oh, feat, preferred_element_type=_f32)
    cnt_ref[...] += jnp.sum(oh, axis=1, keepdims=True)


def _d_body(s_ref, tp_ref, degt_ref, b2_ref, batch_ref, nsum_ref, ncnt_ref):
    @pl.when(pl.program_id(0) == 0)
    def _():
        nsum_ref[...] = jnp.zeros_like(nsum_ref)
        ncnt_ref[...] = jnp.zeros_like(ncnt_ref)

    dinv = _dinv_of(degt_ref[...])
    h2 = (s_ref[0] + s_ref[1] + tp_ref[...]) * dinv + b2_ref[...]
    _pool(batch_ref[0, 0, :], h2, nsum_ref, ncnt_ref)


def _e_body(s_ref, ea_ref, degt_ref, w1_ref, b1_ref, w2_ref, vp_ref):
    dinv = _dinv_of(degt_ref[...])
    u = (s_ref[0] + s_ref[1] + ea_ref[...] * dinv) * dinv
    g1 = jax.nn.sigmoid(jnp.dot(u, w1_ref[...], preferred_element_type=_f32)
                        + b1_ref[...])
    vp_ref[...] = jnp.dot(g1, w2_ref[...], preferred_element_type=_f32) * dinv


def _f_body(s_ref, vp_ref, degt_ref, b2_ref, seg_ref, esum_ref, ecnt_ref):
    @pl.when(pl.program_id(0) == 0)
    def _():
        esum_ref[...] = jnp.zeros_like(esum_ref)
        ecnt_ref[...] = jnp.zeros_like(ecnt_ref)

    dinv = _dinv_of(degt_ref[...])
    g2 = (s_ref[0] + s_ref[1] + vp_ref[...]) * dinv + b2_ref[...]
    _pool(seg_ref[0, 0, :], g2, esum_ref, ecnt_ref)


def _h_body(ea_ref, w1_ref, b1_ref, w2_ref, b2_ref, seg_ref,
            esum_ref, ecnt_ref):
    @pl.when(pl.program_id(0) == 0)
    def _():
        esum_ref[...] = jnp.zeros_like(esum_ref)
        ecnt_ref[...] = jnp.zeros_like(ecnt_ref)

    g1 = jax.nn.sigmoid(jnp.dot(ea_ref[...], w1_ref[...],
                                preferred_element_type=_f32) + b1_ref[...])
    g2 = jnp.dot(g1, w2_ref[...], preferred_element_type=_f32) + b2_ref[...]
    _pool(seg_ref[0, 0, :], g2, esum_ref, ecnt_ref)


def _g_body(ns_ref, nc_ref, el_ref, eh_ref, cl_ref, ch_ref,
            wfn_ref, wfe_ref, bf_ref, out_ref):
    nrepr = ns_ref[...] / jnp.maximum(nc_ref[...], 1.0)
    erepr = ((el_ref[...] + eh_ref[...])
             / jnp.maximum(cl_ref[...] + ch_ref[...], 1.0))
    out_ref[...] = (jnp.dot(nrepr, wfn_ref[...], preferred_element_type=_f32)
                    + jnp.dot(erepr, wfe_ref[...], preferred_element_type=_f32)
                    + bf_ref[...])


def _row_spec(d):
    return pl.BlockSpec((_BLK, d), lambda i: (i, 0))


def _part_spec(d):
    return pl.BlockSpec((2, _BLK, d), lambda i: (0, i, 0))


def _full_spec(shape):
    nd = len(shape)
    return pl.BlockSpec(shape, lambda i: (0,) * nd)


def _seg_spec():
    return pl.BlockSpec((1, 1, _BLK), lambda i: (i, 0, 0))


def _acc_specs(d):
    return [pl.BlockSpec((_G, d), lambda i: (0, 0)),
            pl.BlockSpec((_G, 1), lambda i: (0, 0))]


def kernel(x, edge_index, edge_attr, batch, Wn1, bn1, Wn2, bn2,
           We1, be1, We2, be2, Wf, bf):
    src = edge_index[0].reshape(_NW, _KW, _C)
    dst = edge_index[1].reshape(_NW, _KW, _C)
    zer1 = jnp.zeros((_N,), _f32)
    zer128 = jnp.zeros((_N, 128), _f32)
    zer16 = jnp.zeros((_N, 16), _f32)
    zer32 = jnp.zeros((_N, 32), _f32)

    degp, eseg = _sc_pre()(src, dst, batch, zer1)
    degt = degp.reshape(2, _N).T
    batch3 = batch.reshape(_NB, 1, _BLK)
    eseg_lo3 = eseg[:_N].reshape(_NB, 1, _BLK)
    eseg_hi3 = eseg[_N:].reshape(_HB, 1, _BLK)
    ea_lo = edge_attr[:_N]
    ea_hi = edge_attr[_N:]
    bn1r, bn2r = bn1.reshape(1, -1), bn2.reshape(1, -1)
    be1r, be2r = be1.reshape(1, -1), be2.reshape(1, -1)
    bfr = bf.reshape(1, -1)

    xp, eap = pl.pallas_call(
        _a_body,
        grid=(_NB,),
        in_specs=[_row_spec(2), _row_spec(128), _row_spec(16)],
        out_specs=[_row_spec(128), _row_spec(16)],
        out_shape=[jax.ShapeDtypeStruct((_N, 128), _f32),
                   jax.ShapeDtypeStruct((_N, 16), _f32)],
    )(degt, x, ea_lo)

    s1p = _sc_agg(128)(src, dst, xp, zer128)
    tp = pl.pallas_call(
        _c_body,
        grid=(_NB,),
        in_specs=[_part_spec(128), _row_spec(128), _row_spec(2),
                  _full_spec((128, 256)), _full_spec((1, 256)),
                  _full_spec((256, 128))],
        out_specs=_row_spec(128),
        out_shape=jax.ShapeDtypeStruct((_N, 128), _f32),
    )(s1p, xp, degt, Wn1, bn1r, Wn2)

    s2p = _sc_agg(128)(src, dst, tp, zer128)
    nsum, ncnt = pl.pallas_call(
        _d_body,
        grid=(_NB,),
        in_specs=[_part_spec(128), _row_spec(128), _row_spec(2),
                  _full_spec((1, 128)), _seg_spec()],
        out_specs=_acc_specs(128),
        out_shape=[jax.ShapeDtypeStruct((_G, 128), _f32),
                   jax.ShapeDtypeStruct((_G, 1), _f32)],
    )(s2p, tp, degt, bn2r, batch3)

    se1p = _sc_agg(16)(src, dst, eap, zer16)
    vp = pl.pallas_call(
        _e_body,
        grid=(_NB,),
        in_specs=[_part_spec(16), _row_spec(16), _row_spec(2),
                  _full_spec((16, 64)), _full_spec((1, 64)),
                  _full_spec((64, 32))],
        out_specs=_row_spec(32),
        out_shape=jax.ShapeDtypeStruct((_N, 32), _f32),
    )(se1p, ea_lo, degt, We1, be1r, We2)

    se2p = _sc_agg(32)(src, dst, vp, zer32)
    esum_l, ecnt_l = pl.pallas_call(
        _f_body,
        grid=(_NB,),
        in_specs=[_part_spec(32), _row_spec(32), _row_spec(2),
                  _full_spec((1, 32)), _seg_spec()],
        out_specs=_acc_specs(32),
        out_shape=[jax.ShapeDtypeStruct((_G, 32), _f32),
                   jax.ShapeDtypeStruct((_G, 1), _f32)],
    )(se2p, vp, degt, be2r, eseg_lo3)

    esum_h, ecnt_h = pl.pallas_call(
        _h_body,
        grid=(_HB,),
        in_specs=[_row_spec(16), _full_spec((16, 64)), _full_spec((1, 64)),
                  _full_spec((64, 32)), _full_spec((1, 32)), _seg_spec()],
        out_specs=_acc_specs(32),
        out_shape=[jax.ShapeDtypeStruct((_G, 32), _f32),
                   jax.ShapeDtypeStruct((_G, 1), _f32)],
    )(ea_hi, We1, be1r, We2, be2r, eseg_hi3)

    out = pl.pallas_call(
        _g_body,
        grid=(1,),
        in_specs=[_full_spec((_G, 128)), _full_spec((_G, 1)),
                  _full_spec((_G, 32)), _full_spec((_G, 32)),
                  _full_spec((_G, 1)), _full_spec((_G, 1)),
                  _full_spec((128, 128)), _full_spec((32, 128)),
                  _full_spec((1, 128))],
        out_specs=_full_spec((_G, 128)),
        out_shape=jax.ShapeDtypeStruct((_G, 128), _f32),
    )(nsum, ncnt, esum_l, esum_h, ecnt_l, ecnt_h,
      Wf[:128], Wf[128:], bfr)

    return out


# speedup vs baseline: 91.1061x; 1.1450x over previous
import functools

import jax
import jax.numpy as jnp
from jax import lax
from jax.experimental import pallas as pl
from jax.experimental.pallas import tpu as pltpu
from jax.experimental.pallas import tpu_sc as plsc

_N = 10000
_E = 320000
_G = 64
_C = 80
_NW = 32
_NCH = _E // _C
_KW = _NCH // _NW
_BLK = 1000
_NB = _N // _BLK
_HB = (_E - _N) // _BLK

_f32 = jnp.float32


def _sc_mesh():
    return plsc.VectorSubcoreMesh(core_axis_name="c", subcore_axis_name="s")


def _sc_pre():
    def body(src_hbm, dst_hbm, batch_hbm, zer_hbm, deg_hbm, eseg_hbm,
             src_v, dst_v, seg_v, fbuf_v, ones_v, bat_sh, dacc_sh, sem):
        cid = lax.axis_index("c")
        sid = lax.axis_index("s")
        wid = sid * 2 + cid
        pltpu.sync_copy(src_hbm.at[wid], src_v)
        pltpu.sync_copy(dst_hbm.at[wid], dst_v)
        for i in range(_C // 16):
            ones_v[pl.ds(i * 16, 16)] = jnp.ones((16,), _f32)

        for j in range(8):
            c = pl.multiple_of((sid * 8 + j) * _C, _C)

            @pl.when(sid * 8 + j < 125)
            def _():
                pltpu.sync_copy(batch_hbm.at[pl.ds(c, _C)], seg_v)
                pltpu.sync_copy(seg_v, bat_sh.at[pl.ds(c, _C)])
                pltpu.sync_copy(zer_hbm.at[pl.ds(c, _C)], fbuf_v)
                pltpu.sync_copy(fbuf_v, dacc_sh.at[pl.ds(c, _C)])

        plsc.subcore_barrier()

        def step(k, carry):
            pltpu.async_copy(bat_sh.at[src_v.at[k]], seg_v, sem).wait()
            off = pl.multiple_of((wid * _KW + k) * _C, _C)
            pltpu.sync_copy(seg_v, eseg_hbm.at[pl.ds(off, _C)])
            pltpu.sync_copy(ones_v, dacc_sh.at[dst_v.at[k]], add=True)
            return carry

        lax.fori_loop(0, _KW, step, 0)
        plsc.subcore_barrier()

        for j in range(8):
            c = pl.multiple_of((sid * 8 + j) * _C, _C)
            o = pl.multiple_of(cid * _N + (sid * 8 + j) * _C, _C)

            @pl.when(sid * 8 + j < 125)
            def _():
                pltpu.sync_copy(dacc_sh.at[pl.ds(c, _C)], fbuf_v)
                pltpu.sync_copy(fbuf_v, deg_hbm.at[pl.ds(o, _C)])

    return pl.kernel(
        body,
        out_type=[jax.ShapeDtypeStruct((2 * _N,), _f32),
                  jax.ShapeDtypeStruct((_E,), jnp.int32)],
        mesh=_sc_mesh(),
        compiler_params=pltpu.CompilerParams(use_tc_tiling_on_sc=False),
        scratch_types=[
            pltpu.VMEM((_KW, _C), jnp.int32),
            pltpu.VMEM((_KW, _C), jnp.int32),
            pltpu.VMEM((_C,), jnp.int32),
            pltpu.VMEM((_C,), _f32),
            pltpu.VMEM((_C,), _f32),
            pltpu.VMEM_SHARED((_N,), jnp.int32),
            pltpu.VMEM_SHARED((_N,), _f32),
            pltpu.SemaphoreType.DMA,
        ],
    )


def _sc_agg(d):
    def body(src_hbm, dst_hbm, tab_hbm, zer_hbm, out_hbm,
             src_v, dst_v, b0, b1, b2, acc_sh,
             g0, g1, g2, s0, s1, s2):
        cid = lax.axis_index("c")
        sid = lax.axis_index("s")
        wid = sid * 2 + cid
        bufs = (b0, b1, b2)
        gsem = (g0, g1, g2)
        ssem = (s0, s1, s2)
        pltpu.sync_copy(src_hbm.at[wid], src_v)
        pltpu.sync_copy(dst_hbm.at[wid], dst_v)

        for j in range(8):
            c = pl.multiple_of((sid * 8 + j) * _C, _C)

            @pl.when(sid * 8 + j < 125)
            def _():
                pltpu.sync_copy(zer_hbm.at[pl.ds(c, _C)], b0)
                pltpu.sync_copy(b0, acc_sh.at[pl.ds(c, _C)])

        plsc.subcore_barrier()

        def step(j, carry):
            k = j * 3
            dg = [pltpu.async_copy(tab_hbm.at[src_v.at[k + i]],
                                   bufs[i], gsem[i]) for i in range(3)]
            ds = []
            for i in range(3):
                dg[i].wait()
                ds.append(pltpu.async_copy(bufs[i],
                                           acc_sh.at[dst_v.at[k + i]],
                                           ssem[i], add=True))
            for dd in ds:
                dd.wait()
            return carry

        lax.fori_loop(0, _KW // 3, step, 0)
        for k in range(_KW - _KW % 3, _KW):
            pltpu.async_copy(tab_hbm.at[src_v.at[k]], b0, g0).wait()
            pltpu.sync_copy(b0, acc_sh.at[dst_v.at[k]], add=True)
        plsc.subcore_barrier()

        for j in range(8):
            c = pl.multiple_of((sid * 8 + j) * _C, _C)

            @pl.when(sid * 8 + j < 125)
            def _():
                pltpu.sync_copy(acc_sh.at[pl.ds(c, _C)], b0)
                pltpu.sync_copy(b0, out_hbm.at[cid, pl.ds(c, _C)])

    return pl.kernel(
        body,
        out_type=jax.ShapeDtypeStruct((2, _N, d), _f32),
        mesh=_sc_mesh(),
        compiler_params=pltpu.CompilerParams(use_tc_tiling_on_sc=False),
        scratch_types=[
            pltpu.VMEM((_KW, _C), jnp.int32),
            pltpu.VMEM((_KW, _C), jnp.int32),
            pltpu.VMEM((_C, d), _f32),
            pltpu.VMEM((_C, d), _f32),
            pltpu.VMEM((_C, d), _f32),
            pltpu.VMEM_SHARED((_N, d), _f32),
            pltpu.SemaphoreType.DMA,
            pltpu.SemaphoreType.DMA,
            pltpu.SemaphoreType.DMA,
            pltpu.SemaphoreType.DMA,
            pltpu.SemaphoreType.DMA,
            pltpu.SemaphoreType.DMA,
        ],
    )


def _dinv_of(degt):
    return lax.rsqrt(1.0 + degt[:, 0:1] + degt[:, 1:2])


def _a_body(degt_ref, x_ref, ea_ref, xp_ref, eap_ref):
    dinv = _dinv_of(degt_ref[...])
    xp_ref[...] = x_ref[...] * dinv
    eap_ref[...] = ea_ref[...] * dinv


def _c_body(s_ref, xp_ref, degt_ref, w1_ref, b1_ref, w2_ref, out_ref):
    dinv = _dinv_of(degt_ref[...])
    z = (s_ref[0] + s_ref[1] + xp_ref[...]) * dinv
    h = jax.nn.sigmoid(jnp.dot(z, w1_ref[...], preferred_element_type=_f32)
                       + b1_ref[...])
    out_ref[...] = jnp.dot(h, w2_ref[...], preferred_element_type=_f32) * dinv


def _pool(seg, feat, sum_ref, cnt_ref):
    oh = (lax.broadcasted_iota(jnp.int32, (_G, seg.shape[0]), 0)
          == seg[None, :]).astype(_f32)
    sum_ref[...] += jnp.dot(oh, feat, preferred_element_type=_f32)
    cnt_ref[...] += jnp.sum(oh, axis=1, keepdims=True)


def _d_body(s_ref, tp_ref, degt_ref, b2_ref, batch_ref, nsum_ref, ncnt_ref):
    @pl.when(pl.program_id(0) == 0)
    def _():
        nsum_ref[...] = jnp.zeros_like(nsum_ref)
        ncnt_ref[...] = jnp.zeros_like(ncnt_ref)

    dinv = _dinv_of(degt_ref[...])
    h2 = (s_ref[0] + s_ref[1] + tp_ref[...]) * dinv + b2_ref[...]
    _pool(batch_ref[0, 0, :], h2, nsum_ref, ncnt_ref)


def _e_body(s_ref, ea_ref, degt_ref, w1_ref, b1_ref, w2_ref, vp_ref):
    dinv = _dinv_of(degt_ref[...])
    u = (s_ref[0] + s_ref[1] + ea_ref[...] * dinv) * dinv
    g1 = jax.nn.sigmoid(jnp.dot(u, w1_ref[...], preferred_element_type=_f32)
                        + b1_ref[...])
    vp_ref[...] = jnp.dot(g1, w2_ref[...], preferred_element_type=_f32) * dinv


def _f_body(s_ref, vp_ref, degt_ref, b2_ref, seg_ref, esum_ref, ecnt_ref):
    @pl.when(pl.program_id(0) == 0)
    def _():
        esum_ref[...] = jnp.zeros_like(esum_ref)
        ecnt_ref[...] = jnp.zeros_like(ecnt_ref)

    dinv = _dinv_of(degt_ref[...])
    g2 = (s_ref[0] + s_ref[1] + vp_ref[...]) * dinv + b2_ref[...]
    _pool(seg_ref[0, 0, :], g2, esum_ref, ecnt_ref)


def _h_body(ea_ref, w1_ref, b1_ref, w2_ref, b2_ref, seg_ref,
            esum_ref, ecnt_ref):
    @pl.when(pl.program_id(0) == 0)
    def _():
        esum_ref[...] = jnp.zeros_like(esum_ref)
        ecnt_ref[...] = jnp.zeros_like(ecnt_ref)

    g1 = jax.nn.sigmoid(jnp.dot(ea_ref[...], w1_ref[...],
                                preferred_element_type=_f32) + b1_ref[...])
    g2 = jnp.dot(g1, w2_ref[...], preferred_element_type=_f32) + b2_ref[...]
    _pool(seg_ref[0, 0, :], g2, esum_ref, ecnt_ref)


def _g_body(ns_ref, nc_ref, el_ref, eh_ref, cl_ref, ch_ref,
            wfn_ref, wfe_ref, bf_ref, out_ref):
    nrepr = ns_ref[...] / jnp.maximum(nc_ref[...], 1.0)
    erepr = ((el_ref[...] + eh_ref[...])
             / jnp.maximum(cl_ref[...] + ch_ref[...], 1.0))
    out_ref[...] = (jnp.dot(nrepr, wfn_ref[...], preferred_element_type=_f32)
                    + jnp.dot(erepr, wfe_ref[...], preferred_element_type=_f32)
                    + bf_ref[...])


def _row_spec(d):
    return pl.BlockSpec((_BLK, d), lambda i: (i, 0))


def _part_spec(d):
    return pl.BlockSpec((2, _BLK, d), lambda i: (0, i, 0))


def _full_spec(shape):
    nd = len(shape)
    return pl.BlockSpec(shape, lambda i: (0,) * nd)


def _seg_spec():
    return pl.BlockSpec((1, 1, _BLK), lambda i: (i, 0, 0))


def _acc_specs(d):
    return [pl.BlockSpec((_G, d), lambda i: (0, 0)),
            pl.BlockSpec((_G, 1), lambda i: (0, 0))]


def kernel(x, edge_index, edge_attr, batch, Wn1, bn1, Wn2, bn2,
           We1, be1, We2, be2, Wf, bf):
    src = edge_index[0].reshape(_NW, _KW, _C)
    dst = edge_index[1].reshape(_NW, _KW, _C)
    zer1 = jnp.zeros((_N,), _f32)
    zer128 = jnp.zeros((_N, 128), _f32)
    zer16 = jnp.zeros((_N, 16), _f32)
    zer32 = jnp.zeros((_N, 32), _f32)

    degp, eseg = _sc_pre()(src, dst, batch, zer1)
    degt = degp.reshape(2, _N).T
    batch3 = batch.reshape(_NB, 1, _BLK)
    eseg_lo3 = eseg[:_N].reshape(_NB, 1, _BLK)
    eseg_hi3 = eseg[_N:].reshape(_HB, 1, _BLK)
    ea_lo = edge_attr[:_N]
    ea_hi = edge_attr[_N:]
    bn1r, bn2r = bn1.reshape(1, -1), bn2.reshape(1, -1)
    be1r, be2r = be1.reshape(1, -1), be2.reshape(1, -1)
    bfr = bf.reshape(1, -1)

    xp, eap = pl.pallas_call(
        _a_body,
        grid=(_NB,),
        in_specs=[_row_spec(2), _row_spec(128), _row_spec(16)],
        out_specs=[_row_spec(128), _row_spec(16)],
        out_shape=[jax.ShapeDtypeStruct((_N, 128), _f32),
                   jax.ShapeDtypeStruct((_N, 16), _f32)],
    )(degt, x, ea_lo)

    s1p = _sc_agg(128)(src, dst, xp, zer128)
    tp = pl.pallas_call(
        _c_body,
        grid=(_NB,),
        in_specs=[_part_spec(128), _row_spec(128), _row_spec(2),
                  _full_spec((128, 256)), _full_spec((1, 256)),
                  _full_spec((256, 128))],
        out_specs=_row_spec(128),
        out_shape=jax.ShapeDtypeStruct((_N, 128), _f32),
    )(s1p, xp, degt, Wn1, bn1r, Wn2)

    s2p = _sc_agg(128)(src, dst, tp, zer128)
    nsum, ncnt = pl.pallas_call(
        _d_body,
        grid=(_NB,),
        in_specs=[_part_spec(128), _row_spec(128), _row_spec(2),
                  _full_spec((1, 128)), _seg_spec()],
        out_specs=_acc_specs(128),
        out_shape=[jax.ShapeDtypeStruct((_G, 128), _f32),
                   jax.ShapeDtypeStruct((_G, 1), _f32)],
    )(s2p, tp, degt, bn2r, batch3)

    se1p = _sc_agg(16)(src, dst, eap, zer16)
    vp = pl.pallas_call(
        _e_body,
        grid=(_NB,),
        in_specs=[_part_spec(16), _row_spec(16), _row_spec(2),
                  _full_spec((16, 64)), _full_spec((1, 64)),
                  _full_spec((64, 32))],
        out_specs=_row_spec(32),
        out_shape=jax.ShapeDtypeStruct((_N, 32), _f32),
    )(se1p, ea_lo, degt, We1, be1r, We2)

    se2p = _sc_agg(32)(src, dst, vp, zer32)
    esum_l, ecnt_l = pl.pallas_call(
        _f_body,
        grid=(_NB,),
        in_specs=[_part_spec(32), _row_spec(32), _row_spec(2),
                  _full_spec((1, 32)), _seg_spec()],
        out_specs=_acc_specs(32),
        out_shape=[jax.ShapeDtypeStruct((_G, 32), _f32),
                   jax.ShapeDtypeStruct((_G, 1), _f32)],
    )(se2p, vp, degt, be2r, eseg_lo3)

    esum_h, ecnt_h = pl.pallas_call(
        _h_body,
        grid=(_HB,),
        in_specs=[_row_spec(16), _full_spec((16, 64)), _full_spec((1, 64)),
                  _full_spec((64, 32)), _full_spec((1, 32)), _seg_spec()],
        out_specs=_acc_specs(32),
        out_shape=[jax.ShapeDtypeStruct((_G, 32), _f32),
                   jax.ShapeDtypeStruct((_G, 1), _f32)],
    )(ea_hi, We1, be1r, We2, be2r, eseg_hi3)

    out = pl.pallas_call(
        _g_body,
        grid=(1,),
        in_specs=[_full_spec((_G, 128)), _full_spec((_G, 1)),
                  _full_spec((_G, 32)), _full_spec((_G, 32)),
                  _full_spec((_G, 1)), _full_spec((_G, 1)),
                  _full_spec((128, 128)), _full_spec((32, 128)),
                  _full_spec((1, 128))],
        out_specs=_full_spec((_G, 128)),
        out_shape=jax.ShapeDtypeStruct((_G, 128), _f32),
    )(nsum, ncnt, esum_l, esum_h, ecnt_l, ecnt_h,
      Wf[:128], Wf[128:], bfr)

    return out


# speedup vs baseline: 92.6728x; 1.0172x over previous
import functools

import jax
import jax.numpy as jnp
from jax import lax
from jax.experimental import pallas as pl
from jax.experimental.pallas import tpu as pltpu
from jax.experimental.pallas import tpu_sc as plsc

_N = 10000
_E = 320000
_G = 64
_C = 80
_NW = 32
_NCH = _E // _C
_KW = _NCH // _NW
_BLK = 1000
_NB = _N // _BLK
_HB = (_E - _N) // _BLK

_f32 = jnp.float32


def _sc_mesh():
    return plsc.VectorSubcoreMesh(core_axis_name="c", subcore_axis_name="s")


def _sc_pre():
    def body(src_hbm, dst_hbm, batch_hbm, zer_hbm, deg_hbm, eseg_hbm,
             src_v, dst_v, segs, fbuf_v, ones_v, bat_sh, dacc_sh,
             gsems, wsem, ssem):
        cid = lax.axis_index("c")
        sid = lax.axis_index("s")
        wid = sid * 2 + cid
        pltpu.sync_copy(src_hbm.at[wid], src_v)
        pltpu.sync_copy(dst_hbm.at[wid], dst_v)
        for i in range(_C // 16):
            ones_v[pl.ds(i * 16, 16)] = jnp.ones((16,), _f32)

        for j in range(8):
            c = pl.multiple_of((sid * 8 + j) * _C, _C)

            @pl.when(sid * 8 + j < 125)
            def _():
                pltpu.sync_copy(batch_hbm.at[pl.ds(c, _C)], segs[0])
                pltpu.sync_copy(segs[0], bat_sh.at[pl.ds(c, _C)])
                pltpu.sync_copy(zer_hbm.at[pl.ds(c, _C)], fbuf_v)
                pltpu.sync_copy(fbuf_v, dacc_sh.at[pl.ds(c, _C)])

        plsc.subcore_barrier()

        def step(g, carry):
            k0 = g * 5
            dg = [pltpu.async_copy(bat_sh.at[src_v.at[k0 + i]],
                                   segs[i], gsems[i]) for i in range(5)]
            dws, dss = [], []
            for i in range(5):
                dg[i].wait()
                off = pl.multiple_of((wid * _KW + k0 + i) * _C, _C)
                dws.append(pltpu.async_copy(segs[i],
                                            eseg_hbm.at[pl.ds(off, _C)],
                                            wsem))
                dss.append(pltpu.async_copy(ones_v,
                                            dacc_sh.at[dst_v.at[k0 + i]],
                                            ssem, add=True))
            for dd in dws:
                dd.wait()
            for dd in dss:
                dd.wait()
            return carry

        lax.fori_loop(0, _KW // 5, step, 0)
        plsc.subcore_barrier()

        for j in range(8):
            c = pl.multiple_of((sid * 8 + j) * _C, _C)
            o = pl.multiple_of(cid * _N + (sid * 8 + j) * _C, _C)

            @pl.when(sid * 8 + j < 125)
            def _():
                pltpu.sync_copy(dacc_sh.at[pl.ds(c, _C)], fbuf_v)
                pltpu.sync_copy(fbuf_v, deg_hbm.at[pl.ds(o, _C)])

    return pl.kernel(
        body,
        out_type=[jax.ShapeDtypeStruct((2 * _N,), _f32),
                  jax.ShapeDtypeStruct((_E,), jnp.int32)],
        mesh=_sc_mesh(),
        compiler_params=pltpu.CompilerParams(use_tc_tiling_on_sc=True),
        scratch_types=[
            pltpu.VMEM((_KW, _C), jnp.int32),
            pltpu.VMEM((_KW, _C), jnp.int32),
            [pltpu.VMEM((_C,), jnp.int32) for _ in range(5)],
            pltpu.VMEM((_C,), _f32),
            pltpu.VMEM((_C,), _f32),
            pltpu.VMEM_SHARED((_N,), jnp.int32),
            pltpu.VMEM_SHARED((_N,), _f32),
            [pltpu.SemaphoreType.DMA for _ in range(5)],
            pltpu.SemaphoreType.DMA,
            pltpu.SemaphoreType.DMA,
        ],
    )


def _sc_agg(d, tc_tiling):
    depth = 4 if d >= 128 else 8
    ngrp = 128 // depth

    def body(src_hbm, dst_hbm, tab_hbm, zer_hbm, out_hbm,
             sA, dA, sB, dB, bufs, acc_sh, gi, gsem, ssem):
        cid = lax.axis_index("c")
        sid = lax.axis_index("s")
        wid = sid * 2 + cid

        for j in range(8):
            c = pl.multiple_of((sid * 8 + j) * _C, _C)

            @pl.when(sid * 8 + j < 125)
            def _():
                pltpu.sync_copy(zer_hbm.at[pl.ds(c, _C)], bufs[0])
                pltpu.sync_copy(bufs[0], acc_sh.at[pl.ds(c, _C)])

        plsc.subcore_barrier()

        def process(sx, dx):
            dg = [pltpu.async_copy(tab_hbm.at[sx.at[i]], bufs[i], gsem[i])
                  for i in range(depth)]
            ds = []
            for i in range(depth):
                dg[i].wait()
                ds.append(pltpu.async_copy(bufs[i], acc_sh.at[dx.at[i]],
                                           ssem[i], add=True))
            for dd in ds:
                dd.wait()

        pltpu.sync_copy(src_hbm.at[wid, 0], sA)
        pltpu.sync_copy(dst_hbm.at[wid, 0], dA)

        def pair(m, carry):
            g0 = m * 2
            ia = pltpu.async_copy(src_hbm.at[wid, g0 + 1], sB, gi)
            ib = pltpu.async_copy(dst_hbm.at[wid, g0 + 1], dB, gi)
            process(sA, dA)
            ia.wait()
            ib.wait()

            @pl.when(g0 + 2 < ngrp)
            def _():
                pltpu.async_copy(src_hbm.at[wid, g0 + 2], sA, gi)
                pltpu.async_copy(dst_hbm.at[wid, g0 + 2], dA, gi)

            process(sB, dB)

            @pl.when(g0 + 2 < ngrp)
            def _():
                pltpu.make_async_copy(src_hbm.at[wid, g0 + 2], sA, gi).wait()
                pltpu.make_async_copy(dst_hbm.at[wid, g0 + 2], dA, gi).wait()

            return carry

        lax.fori_loop(0, ngrp // 2, pair, 0)
        plsc.subcore_barrier()

        for j in range(8):
            c = pl.multiple_of((sid * 8 + j) * _C, _C)

            @pl.when(sid * 8 + j < 125)
            def _():
                pltpu.sync_copy(acc_sh.at[pl.ds(c, _C)], bufs[0])
                pltpu.sync_copy(bufs[0], out_hbm.at[cid, pl.ds(c, _C)])

    return pl.kernel(
        body,
        out_type=jax.ShapeDtypeStruct((2, _N, d), _f32),
        mesh=_sc_mesh(),
        compiler_params=pltpu.CompilerParams(use_tc_tiling_on_sc=tc_tiling),
        scratch_types=[
            pltpu.VMEM((depth, _C), jnp.int32),
            pltpu.VMEM((depth, _C), jnp.int32),
            pltpu.VMEM((depth, _C), jnp.int32),
            pltpu.VMEM((depth, _C), jnp.int32),
            [pltpu.VMEM((_C, d), _f32) for _ in range(depth)],
            pltpu.VMEM_SHARED((_N + _C, d), _f32),
            pltpu.SemaphoreType.DMA,
            [pltpu.SemaphoreType.DMA for _ in range(depth)],
            [pltpu.SemaphoreType.DMA for _ in range(depth)],
        ],
    )


def _dinv_of(degt):
    return lax.rsqrt(1.0 + degt[:, 0:1] + degt[:, 1:2])


def _a_body(degt_ref, x_ref, ea_ref, xp_ref, eap_ref):
    dinv = _dinv_of(degt_ref[...])
    xp_ref[...] = x_ref[...] * dinv
    eap_ref[...] = ea_ref[...] * dinv


def _c_body(s_ref, xp_ref, degt_ref, w1_ref, b1_ref, w2_ref, out_ref):
    dinv = _dinv_of(degt_ref[...])
    z = (s_ref[0] + s_ref[1] + xp_ref[...]) * dinv
    h = jax.nn.sigmoid(jnp.dot(z, w1_ref[...], preferred_element_type=_f32)
                       + b1_ref[...])
    out_ref[...] = jnp.dot(h, w2_ref[...], preferred_element_type=_f32) * dinv


def _pool(seg, feat, sum_ref, cnt_ref):
    oh = (lax.broadcasted_iota(jnp.int32, (_G, seg.shape[0]), 0)
          == seg[None, :]).astype(_f32)
    sum_ref[...] += jnp.dot(oh, feat, preferred_element_type=_f32)
    cnt_ref[...] += jnp.sum(oh, axis=1, keepdims=True)


def _d_body(s_ref, tp_ref, degt_ref, b2_ref, batch_ref, nsum_ref, ncnt_ref):
    @pl.when(pl.program_id(0) == 0)
    def _():
        nsum_ref[...] = jnp.zeros_like(nsum_ref)
        ncnt_ref[...] = jnp.zeros_like(ncnt_ref)

    dinv = _dinv_of(degt_ref[...])
    h2 = (s_ref[0] + s_ref[1] + tp_ref[...]) * dinv + b2_ref[...]
    _pool(batch_ref[0, 0, :], h2, nsum_ref, ncnt_ref)


def _e_body(s_ref, ea_ref, degt_ref, w1_ref, b1_ref, w2_ref, vp_ref):
    dinv = _dinv_of(degt_ref[...])
    u = (s_ref[0] + s_ref[1] + ea_ref[...] * dinv) * dinv
    g1 = jax.nn.sigmoid(jnp.dot(u, w1_ref[...], preferred_element_type=_f32)
                        + b1_ref[...])
    vp_ref[...] = jnp.dot(g1, w2_ref[...], preferred_element_type=_f32) * dinv


def _f_body(s_ref, vp_ref, degt_ref, b2_ref, seg_ref, esum_ref, ecnt_ref):
    @pl.when(pl.program_id(0) == 0)
    def _():
        esum_ref[...] = jnp.zeros_like(esum_ref)
        ecnt_ref[...] = jnp.zeros_like(ecnt_ref)

    dinv = _dinv_of(degt_ref[...])
    g2 = (s_ref[0] + s_ref[1] + vp_ref[...]) * dinv + b2_ref[...]
    _pool(seg_ref[0, 0, :], g2, esum_ref, ecnt_ref)


def _h_body(ea_ref, w1_ref, b1_ref, w2_ref, b2_ref, seg_ref,
            esum_ref, ecnt_ref):
    @pl.when(pl.program_id(0) == 0)
    def _():
        esum_ref[...] = jnp.zeros_like(esum_ref)
        ecnt_ref[...] = jnp.zeros_like(ecnt_ref)

    g1 = jax.nn.sigmoid(jnp.dot(ea_ref[...], w1_ref[...],
                                preferred_element_type=_f32) + b1_ref[...])
    g2 = jnp.dot(g1, w2_ref[...], preferred_element_type=_f32) + b2_ref[...]
    _pool(seg_ref[0, 0, :], g2, esum_ref, ecnt_ref)


def _g_body(ns_ref, nc_ref, el_ref, eh_ref, cl_ref, ch_ref,
            wfn_ref, wfe_ref, bf_ref, out_ref):
    nrepr = ns_ref[...] / jnp.maximum(nc_ref[...], 1.0)
    erepr = ((el_ref[...] + eh_ref[...])
             / jnp.maximum(cl_ref[...] + ch_ref[...], 1.0))
    out_ref[...] = (jnp.dot(nrepr, wfn_ref[...], preferred_element_type=_f32)
                    + jnp.dot(erepr, wfe_ref[...], preferred_element_type=_f32)
                    + bf_ref[...])


def _row_spec(d):
    return pl.BlockSpec((_BLK, d), lambda i: (i, 0))


def _part_spec(d):
    return pl.BlockSpec((2, _BLK, d), lambda i: (0, i, 0))


def _full_spec(shape):
    nd = len(shape)
    return pl.BlockSpec(shape, lambda i: (0,) * nd)


def _seg_spec():
    return pl.BlockSpec((1, 1, _BLK), lambda i: (i, 0, 0))


def _acc_specs(d):
    return [pl.BlockSpec((_G, d), lambda i: (0, 0)),
            pl.BlockSpec((_G, 1), lambda i: (0, 0))]


def kernel(x, edge_index, edge_attr, batch, Wn1, bn1, Wn2, bn2,
           We1, be1, We2, be2, Wf, bf):
    src = edge_index[0].reshape(_NW, _KW, _C)
    dst = edge_index[1].reshape(_NW, _KW, _C)
    lane = jnp.arange(_C, dtype=jnp.int32)
    src_pad = jnp.concatenate(
        [src, jnp.broadcast_to(lane, (_NW, 3, _C))], axis=1)
    dst_pad = jnp.concatenate(
        [dst, jnp.broadcast_to(_N + lane, (_NW, 3, _C))], axis=1)
    src4 = src_pad.reshape(_NW, 32, 4, _C)
    dst4 = dst_pad.reshape(_NW, 32, 4, _C)
    src8 = src_pad.reshape(_NW, 16, 8, _C)
    dst8 = dst_pad.reshape(_NW, 16, 8, _C)
    zer1 = jnp.zeros((_N,), _f32)
    zer128 = jnp.zeros((_N, 128), _f32)
    zer16 = jnp.zeros((_N, 16), _f32)
    zer32 = jnp.zeros((_N, 32), _f32)

    degp, eseg = _sc_pre()(src, dst, batch, zer1)
    degt = degp.reshape(2, _N).T
    batch3 = batch.reshape(_NB, 1, _BLK)
    eseg_lo3 = eseg[:_N].reshape(_NB, 1, _BLK)
    eseg_hi3 = eseg[_N:].reshape(_HB, 1, _BLK)
    ea_lo = edge_attr[:_N]
    ea_hi = edge_attr[_N:]
    bn1r, bn2r = bn1.reshape(1, -1), bn2.reshape(1, -1)
    be1r, be2r = be1.reshape(1, -1), be2.reshape(1, -1)
    bfr = bf.reshape(1, -1)

    xp, eap = pl.pallas_call(
        _a_body,
        grid=(_NB,),
        in_specs=[_row_spec(2), _row_spec(128), _row_spec(16)],
        out_specs=[_row_spec(128), _row_spec(16)],
        out_shape=[jax.ShapeDtypeStruct((_N, 128), _f32),
                   jax.ShapeDtypeStruct((_N, 16), _f32)],
    )(degt, x, ea_lo)

    s1p = _sc_agg(128, True)(src4, dst4, xp, zer128)
    tp = pl.pallas_call(
        _c_body,
        grid=(_NB,),
        in_specs=[_part_spec(128), _row_spec(128), _row_spec(2),
                  _full_spec((128, 256)), _full_spec((1, 256)),
                  _full_spec((256, 128))],
        out_specs=_row_spec(128),
        out_shape=jax.ShapeDtypeStruct((_N, 128), _f32),
    )(s1p, xp, degt, Wn1, bn1r, Wn2)

    s2p = _sc_agg(128, True)(src4, dst4, tp, zer128)
    nsum, ncnt = pl.pallas_call(
        _d_body,
        grid=(_NB,),
        in_specs=[_part_spec(128), _row_spec(128), _row_spec(2),
                  _full_spec((1, 128)), _seg_spec()],
        out_specs=_acc_specs(128),
        out_shape=[jax.ShapeDtypeStruct((_G, 128), _f32),
                   jax.ShapeDtypeStruct((_G, 1), _f32)],
    )(s2p, tp, degt, bn2r, batch3)

    se1p = _sc_agg(16, False)(src8, dst8, eap, zer16)
    vp = pl.pallas_call(
        _e_body,
        grid=(_NB,),
        in_specs=[_part_spec(16), _row_spec(16), _row_spec(2),
                  _full_spec((16, 64)), _full_spec((1, 64)),
                  _full_spec((64, 32))],
        out_specs=_row_spec(32),
        out_shape=jax.ShapeDtypeStruct((_N, 32), _f32),
    )(se1p, ea_lo, degt, We1, be1r, We2)

    se2p = _sc_agg(32, False)(src8, dst8, vp, zer32)
    esum_l, ecnt_l = pl.pallas_call(
        _f_body,
        grid=(_NB,),
        in_specs=[_part_spec(32), _row_spec(32), _row_spec(2),
                  _full_spec((1, 32)), _seg_spec()],
        out_specs=_acc_specs(32),
        out_shape=[jax.ShapeDtypeStruct((_G, 32), _f32),
                   jax.ShapeDtypeStruct((_G, 1), _f32)],
    )(se2p, vp, degt, be2r, eseg_lo3)

    esum_h, ecnt_h = pl.pallas_call(
        _h_body,
        grid=(_HB,),
        in_specs=[_row_spec(16), _full_spec((16, 64)), _full_spec((1, 64)),
                  _full_spec((64, 32)), _full_spec((1, 32)), _seg_spec()],
        out_specs=_acc_specs(32),
        out_shape=[jax.ShapeDtypeStruct((_G, 32), _f32),
                   jax.ShapeDtypeStruct((_G, 1), _f32)],
    )(ea_hi, We1, be1r, We2, be2r, eseg_hi3)

    out = pl.pallas_call(
        _g_body,
        grid=(1,),
        in_specs=[_full_spec((_G, 128)), _full_spec((_G, 1)),
                  _full_spec((_G, 32)), _full_spec((_G, 32)),
                  _full_spec((_G, 1)), _full_spec((_G, 1)),
                  _full_spec((128, 128)), _full_spec((32, 128)),
                  _full_spec((1, 128))],
        out_specs=_full_spec((_G, 128)),
        out_shape=jax.ShapeDtypeStruct((_G, 128), _f32),
    )(nsum, ncnt, esum_l, esum_h, ecnt_l, ecnt_h,
      Wf[:128], Wf[128:], bfr)

    return out
